# Initial kernel scaffold; baseline (speedup 1.0000x reference)
import jax
import jax.numpy as jnp
from jax.experimental import pallas as pl


def kernel(edge_index, edge_type, embed_w, params):
    raise NotImplementedError("write your pallas kernel here")


# speedup vs baseline: 1.9697x; 1.9697x over previous
import functools

import jax
import jax.numpy as jnp
from jax import lax
from jax.experimental import pallas as pl
from jax.experimental.pallas import tpu as pltpu
from jax.experimental.pallas import tpu_sc as plsc


N = 10000
E = 320000
D = 128
R = 64

NC = 2
NS = 16
NW = NC * NS
EPW = E // NW
K = 80
NCH = EPW // K
NP = 10240
RPS = NP // NS

_NBLK = 10
_BN = N // _NBLK


def _gb_kernel(emb_ref, w1_ref, b1_ref, w2_ref, b2_ref, out_ref):
    h = jnp.maximum(
        jnp.dot(emb_ref[...], w1_ref[...], preferred_element_type=jnp.float32)
        + b1_ref[...], 0.0)
    out_ref[...] = (
        jnp.dot(h, w2_ref[...], preferred_element_type=jnp.float32)
        + b2_ref[...])


def _gb_tables(p):
    return pl.pallas_call(
        _gb_kernel,
        out_shape=jax.ShapeDtypeStruct((R, 2 * D), jnp.float32),
    )(p["rel_emb"], p["rel_w1"], p["rel_b1"].reshape(1, 2 * D),
      p["rel_w2"], p["rel_b2"].reshape(1, 2 * D))


def _lin_kernel(x_ref, w_ref, b_ref, out_ref):
    out_ref[...] = (
        jnp.dot(x_ref[...], w_ref[...], preferred_element_type=jnp.float32)
        + b_ref[...])


def _linear(x, w, b):
    return pl.pallas_call(
        _lin_kernel,
        grid=(_NBLK,),
        in_specs=[
            pl.BlockSpec((_BN, D), lambda i: (i, 0)),
            pl.BlockSpec((D, D), lambda i: (0, 0)),
            pl.BlockSpec((1, D), lambda i: (0, 0)),
        ],
        out_specs=pl.BlockSpec((_BN, D), lambda i: (i, 0)),
        out_shape=jax.ShapeDtypeStruct((N, D), jnp.float32),
    )(x, w, b.reshape(1, D))


def _upd_kernel(x_ref, a0_ref, a1_ref, eps_ref, w1_ref, b1_ref, w2_ref,
                b2_ref, out_ref):
    z = (1.0 + eps_ref[0, 0]) * x_ref[...] + a0_ref[...] + a1_ref[...]
    h = jnp.maximum(
        jnp.dot(z, w1_ref[...], preferred_element_type=jnp.float32)
        + b1_ref[...], 0.0)
    out_ref[...] = (
        jnp.dot(h, w2_ref[...], preferred_element_type=jnp.float32)
        + b2_ref[...])


def _update(x, a0, a1, p):
    return pl.pallas_call(
        _upd_kernel,
        grid=(_NBLK,),
        in_specs=[
            pl.BlockSpec((_BN, D), lambda i: (i, 0)),
            pl.BlockSpec((_BN, D), lambda i: (i, 0)),
            pl.BlockSpec((_BN, D), lambda i: (i, 0)),
            pl.BlockSpec((1, 1), lambda i: (0, 0)),
            pl.BlockSpec((D, 2 * D), lambda i: (0, 0)),
            pl.BlockSpec((1, 2 * D), lambda i: (0, 0)),
            pl.BlockSpec((2 * D, D), lambda i: (0, 0)),
            pl.BlockSpec((1, D), lambda i: (0, 0)),
        ],
        out_specs=pl.BlockSpec((_BN, D), lambda i: (i, 0)),
        out_shape=jax.ShapeDtypeStruct((N, D), jnp.float32),
    )(x, a0, a1, p["eps"].reshape(1, 1), p["mlp_w1"],
      p["mlp_b1"].reshape(1, 2 * D), p["mlp_w2"], p["mlp_b2"].reshape(1, D))


def _edge_body(src_hbm, dst_hbm, typ_hbm, y_hbm, gb_hbm, out_hbm,
               src_v, dst_v, typ_v, rows_v, gb_v, prod_v, acc,
               sem1, sem2):
    c = lax.axis_index("c")
    s = lax.axis_index("s")
    wid = c * NS + s
    base = wid * EPW

    def zrow(i, _):
        for j in range(D // 16):
            prod_v[i, pl.ds(j * 16, 16)] = jnp.zeros((16,), jnp.float32)
        return 0
    lax.fori_loop(0, K, zrow, 0)
    for k in range(RPS // K):
        pltpu.sync_copy(prod_v, acc.at[pl.ds(s * RPS + k * K, K)])
    plsc.subcore_barrier()

    def chunk(ci, _):
        off = base + ci * K
        pltpu.sync_copy(src_hbm.at[pl.ds(off, K)], src_v)
        pltpu.sync_copy(dst_hbm.at[pl.ds(off, K)], dst_v)
        pltpu.sync_copy(typ_hbm.at[pl.ds(off, K)], typ_v)
        cp1 = pltpu.async_copy(y_hbm.at[src_v], rows_v, sem1)
        cp2 = pltpu.async_copy(gb_hbm.at[typ_v], gb_v, sem2)
        cp1.wait()
        cp2.wait()

        def erow(e, _):
            for j in range(D // 16):
                sl = pl.ds(j * 16, 16)
                prod_v[e, sl] = (rows_v[e, sl] * gb_v[e, sl]
                                 + gb_v[e, pl.ds(D + j * 16, 16)])
            return 0
        lax.fori_loop(0, K, erow, 0)
        pltpu.sync_copy(prod_v, acc.at[dst_v], add=True)
        return 0
    lax.fori_loop(0, NCH, chunk, 0)

    plsc.subcore_barrier()
    for k in range(RPS // K):
        r0 = s * RPS + k * K
        pltpu.sync_copy(acc.at[pl.ds(r0, K)], out_hbm.at[c, pl.ds(r0, K)])


@functools.partial(jax.jit, static_argnums=())
def _edge_pass(src, dst, typ, y, gb):
    f = pl.kernel(
        _edge_body,
        out_type=jax.ShapeDtypeStruct((NC, NP, D), jnp.float32),
        mesh=plsc.VectorSubcoreMesh(core_axis_name="c", subcore_axis_name="s"),
        scratch_types=[
            pltpu.VMEM((K,), jnp.int32),
            pltpu.VMEM((K,), jnp.int32),
            pltpu.VMEM((K,), jnp.int32),
            pltpu.VMEM((K, D), jnp.float32),
            pltpu.VMEM((K, 2 * D), jnp.float32),
            pltpu.VMEM((K, D), jnp.float32),
            pltpu.VMEM_SHARED((NP, D), jnp.float32),
            pltpu.SemaphoreType.DMA,
            pltpu.SemaphoreType.DMA,
        ],
    )
    return f(src, dst, typ, y, gb)


def kernel(edge_index, edge_type, embed_w, params):
    src = edge_index[0]
    dst = edge_index[1]
    x = embed_w
    for p in params:
        gb = _gb_tables(p)
        y = _linear(x, p["W_w"], p["W_b"])
        part = _edge_pass(src, dst, edge_type, y, gb)
        x = _update(x, part[0, :N], part[1, :N], p)
    return x


# speedup vs baseline: 2.9759x; 1.5108x over previous
import functools

import jax
import jax.numpy as jnp
from jax import lax
from jax.experimental import pallas as pl
from jax.experimental.pallas import tpu as pltpu
from jax.experimental.pallas import tpu_sc as plsc


N = 10000
E = 320000
D = 128
R = 64

NC = 2
NS = 16
NW = NC * NS
EPW = E // NW
K = 40
NCH = EPW // K
NP = 10240
RPS = NP // NS
NBLK = 25
CPB = 10
EPB = CPB * K

_NBLK = 10
_BN = N // _NBLK


def _gb_kernel(emb_ref, w1_ref, b1_ref, w2_ref, b2_ref, out_ref):
    h = jnp.maximum(
        jnp.dot(emb_ref[...], w1_ref[...], preferred_element_type=jnp.float32)
        + b1_ref[...], 0.0)
    out_ref[...] = (
        jnp.dot(h, w2_ref[...], preferred_element_type=jnp.float32)
        + b2_ref[...])


def _gb_tables(p):
    return pl.pallas_call(
        _gb_kernel,
        out_shape=jax.ShapeDtypeStruct((R, 2 * D), jnp.float32),
    )(p["rel_emb"], p["rel_w1"], p["rel_b1"].reshape(1, 2 * D),
      p["rel_w2"], p["rel_b2"].reshape(1, 2 * D))


def _lin_kernel(x_ref, w_ref, b_ref, out_ref):
    out_ref[...] = (
        jnp.dot(x_ref[...], w_ref[...], preferred_element_type=jnp.float32)
        + b_ref[...])


def _linear(x, w, b):
    return pl.pallas_call(
        _lin_kernel,
        grid=(_NBLK,),
        in_specs=[
            pl.BlockSpec((_BN, D), lambda i: (i, 0)),
            pl.BlockSpec((D, D), lambda i: (0, 0)),
            pl.BlockSpec((1, D), lambda i: (0, 0)),
        ],
        out_specs=pl.BlockSpec((_BN, D), lambda i: (i, 0)),
        out_shape=jax.ShapeDtypeStruct((N, D), jnp.float32),
    )(x, w, b.reshape(1, D))


def _upd_kernel(x_ref, a0_ref, a1_ref, eps_ref, w1_ref, b1_ref, w2_ref,
                b2_ref, out_ref):
    z = (1.0 + eps_ref[0, 0]) * x_ref[...] + a0_ref[...] + a1_ref[...]
    h = jnp.maximum(
        jnp.dot(z, w1_ref[...], preferred_element_type=jnp.float32)
        + b1_ref[...], 0.0)
    out_ref[...] = (
        jnp.dot(h, w2_ref[...], preferred_element_type=jnp.float32)
        + b2_ref[...])


def _update(x, a0, a1, p):
    return pl.pallas_call(
        _upd_kernel,
        grid=(_NBLK,),
        in_specs=[
            pl.BlockSpec((_BN, D), lambda i: (i, 0)),
            pl.BlockSpec((_BN, D), lambda i: (i, 0)),
            pl.BlockSpec((_BN, D), lambda i: (i, 0)),
            pl.BlockSpec((1, 1), lambda i: (0, 0)),
            pl.BlockSpec((D, 2 * D), lambda i: (0, 0)),
            pl.BlockSpec((1, 2 * D), lambda i: (0, 0)),
            pl.BlockSpec((2 * D, D), lambda i: (0, 0)),
            pl.BlockSpec((1, D), lambda i: (0, 0)),
        ],
        out_specs=pl.BlockSpec((_BN, D), lambda i: (i, 0)),
        out_shape=jax.ShapeDtypeStruct((N, D), jnp.float32),
    )(x, a0, a1, p["eps"].reshape(1, 1), p["mlp_w1"],
      p["mlp_b1"].reshape(1, 2 * D), p["mlp_w2"], p["mlp_b2"].reshape(1, D))


def _edge_body(src_hbm, dst3_hbm, typ_hbm, y_hbm, gb_hbm, out_hbm,
               srcblk, typblk, dstblk,
               rows0, rows1, gbv0, gbv1, prod0, prod1, acc,
               semy0, semy1, semg0, semg1, sems0, sems1):
    rows = (rows0, rows1)
    gbv = (gbv0, gbv1)
    prod = (prod0, prod1)
    semy = (semy0, semy1)
    semg = (semg0, semg1)
    sems = (sems0, sems1)

    c = lax.axis_index("c")
    s = lax.axis_index("s")
    wid = c * NS + s
    base = wid * EPW

    def zrow(i, _):
        for j in range(D // 16):
            prod0[i, pl.ds(j * 16, 16)] = jnp.zeros((16,), jnp.float32)
        return 0
    lax.fori_loop(0, K, zrow, 0)
    for k in range(RPS // K):
        pltpu.sync_copy(prod0, acc.at[pl.ds(s * RPS + k * K, K)])
    plsc.subcore_barrier()

    def fire(b, j):
        pltpu.async_copy(y_hbm.at[srcblk.at[pl.ds(j * K, K)]], rows[b],
                         semy[b])
        pltpu.async_copy(gb_hbm.at[typblk.at[pl.ds(j * K, K)]], gbv[b],
                         semg[b])

    def wait_gathers(b):
        pltpu.make_async_copy(y_hbm.at[srcblk.at[pl.ds(0, K)]], rows[b],
                              semy[b]).wait()
        pltpu.make_async_copy(gb_hbm.at[typblk.at[pl.ds(0, K)]], gbv[b],
                              semg[b]).wait()

    def wait_scatter(b):
        pltpu.make_async_copy(prod[b], acc.at[dstblk.at[0]], sems[b]).wait()

    def block(g, _):
        @pl.when(g > 0)
        def _():
            wait_scatter(0)
            wait_scatter(1)
        bb = base + g * EPB
        pltpu.sync_copy(src_hbm.at[pl.ds(bb, EPB)], srcblk)
        pltpu.sync_copy(typ_hbm.at[pl.ds(bb, EPB)], typblk)
        pltpu.sync_copy(dst3_hbm.at[wid * NBLK + g], dstblk)
        fire(0, 0)
        fire(1, 1)

        def pair(pp, _):
            for b in range(2):
                j = 2 * pp + b
                wait_gathers(b)

                @pl.when(pp >= 1)
                def _():
                    wait_scatter(b)

                def erow(e, _):
                    for q in range(D // 16):
                        sl = pl.ds(q * 16, 16)
                        prod[b][e, sl] = (rows[b][e, sl] * gbv[b][e, sl]
                                          + gbv[b][e, pl.ds(D + q * 16, 16)])
                    return 0
                lax.fori_loop(0, K, erow, 0)
                pltpu.async_copy(prod[b], acc.at[dstblk.at[j]], sems[b],
                                 add=True)

                @pl.when(pp <= CPB // 2 - 2)
                def _():
                    fire(b, j + 2)
            return 0
        lax.fori_loop(0, CPB // 2, pair, 0)
        return 0
    lax.fori_loop(0, NBLK, block, 0)
    wait_scatter(0)
    wait_scatter(1)

    plsc.subcore_barrier()
    for k in range(RPS // K):
        r0 = s * RPS + k * K
        pltpu.sync_copy(acc.at[pl.ds(r0, K)], out_hbm.at[c, pl.ds(r0, K)])


@functools.partial(jax.jit, static_argnums=())
def _edge_pass(src, dst3, typ, y, gb):
    f = pl.kernel(
        _edge_body,
        out_type=jax.ShapeDtypeStruct((NC, NP, D), jnp.float32),
        mesh=plsc.VectorSubcoreMesh(core_axis_name="c", subcore_axis_name="s"),
        scratch_types=[
            pltpu.VMEM((EPB,), jnp.int32),
            pltpu.VMEM((EPB,), jnp.int32),
            pltpu.VMEM((CPB, K), jnp.int32),
            pltpu.VMEM((K, D), jnp.float32),
            pltpu.VMEM((K, D), jnp.float32),
            pltpu.VMEM((K, 2 * D), jnp.float32),
            pltpu.VMEM((K, 2 * D), jnp.float32),
            pltpu.VMEM((K, D), jnp.float32),
            pltpu.VMEM((K, D), jnp.float32),
            pltpu.VMEM_SHARED((NP, D), jnp.float32),
            pltpu.SemaphoreType.DMA,
            pltpu.SemaphoreType.DMA,
            pltpu.SemaphoreType.DMA,
            pltpu.SemaphoreType.DMA,
            pltpu.SemaphoreType.DMA,
            pltpu.SemaphoreType.DMA,
        ],
    )
    return f(src, dst3, typ, y, gb)


def kernel(edge_index, edge_type, embed_w, params):
    src = edge_index[0]
    dst3 = edge_index[1].reshape(NW * NBLK, CPB, K)
    x = embed_w
    for p in params:
        gb = _gb_tables(p)
        y = _linear(x, p["W_w"], p["W_b"])
        part = _edge_pass(src, dst3, edge_type, y, gb)
        x = _update(x, part[0, :N], part[1, :N], p)
    return x


# speedup vs baseline: 3.6999x; 1.2433x over previous
import functools

import jax
import jax.numpy as jnp
from jax import lax
from jax.experimental import pallas as pl
from jax.experimental.pallas import tpu as pltpu
from jax.experimental.pallas import tpu_sc as plsc


N = 10000
E = 320000
D = 128
R = 64

NC = 2
NS = 16
NW = NC * NS
EPW = E // NW
K = 40
NCH = EPW // K
NP = 10240
RPS = NP // NS
NBLK = 25
CPB = 10
EPB = CPB * K

_NBLK = 10
_BN = N // _NBLK


def _gb_kernel(emb_ref, w1_ref, b1_ref, w2_ref, b2_ref, out_ref):
    h = jnp.maximum(
        jnp.dot(emb_ref[...], w1_ref[...], preferred_element_type=jnp.float32)
        + b1_ref[...], 0.0)
    out_ref[...] = (
        jnp.dot(h, w2_ref[...], preferred_element_type=jnp.float32)
        + b2_ref[...])


def _gb_tables(p):
    return pl.pallas_call(
        _gb_kernel,
        out_shape=jax.ShapeDtypeStruct((R, 2 * D), jnp.float32),
    )(p["rel_emb"], p["rel_w1"], p["rel_b1"].reshape(1, 2 * D),
      p["rel_w2"], p["rel_b2"].reshape(1, 2 * D))


def _lin_kernel(x_ref, w_ref, b_ref, out_ref):
    out_ref[...] = (
        jnp.dot(x_ref[...], w_ref[...], preferred_element_type=jnp.float32)
        + b_ref[...])


def _linear(x, w, b):
    return pl.pallas_call(
        _lin_kernel,
        grid=(_NBLK,),
        in_specs=[
            pl.BlockSpec((_BN, D), lambda i: (i, 0)),
            pl.BlockSpec((D, D), lambda i: (0, 0)),
            pl.BlockSpec((1, D), lambda i: (0, 0)),
        ],
        out_specs=pl.BlockSpec((_BN, D), lambda i: (i, 0)),
        out_shape=jax.ShapeDtypeStruct((N, D), jnp.float32),
    )(x, w, b.reshape(1, D))


def _upd_kernel(x_ref, a0_ref, a1_ref, eps_ref, w1_ref, b1_ref, w2_ref,
                b2_ref, out_ref):
    z = (1.0 + eps_ref[0, 0]) * x_ref[...] + a0_ref[...] + a1_ref[...]
    h = jnp.maximum(
        jnp.dot(z, w1_ref[...], preferred_element_type=jnp.float32)
        + b1_ref[...], 0.0)
    out_ref[...] = (
        jnp.dot(h, w2_ref[...], preferred_element_type=jnp.float32)
        + b2_ref[...])


def _update(x, a0, a1, p):
    return pl.pallas_call(
        _upd_kernel,
        grid=(_NBLK,),
        in_specs=[
            pl.BlockSpec((_BN, D), lambda i: (i, 0)),
            pl.BlockSpec((_BN, D), lambda i: (i, 0)),
            pl.BlockSpec((_BN, D), lambda i: (i, 0)),
            pl.BlockSpec((1, 1), lambda i: (0, 0)),
            pl.BlockSpec((D, 2 * D), lambda i: (0, 0)),
            pl.BlockSpec((1, 2 * D), lambda i: (0, 0)),
            pl.BlockSpec((2 * D, D), lambda i: (0, 0)),
            pl.BlockSpec((1, D), lambda i: (0, 0)),
        ],
        out_specs=pl.BlockSpec((_BN, D), lambda i: (i, 0)),
        out_shape=jax.ShapeDtypeStruct((N, D), jnp.float32),
    )(x, a0, a1, p["eps"].reshape(1, 1), p["mlp_w1"],
      p["mlp_b1"].reshape(1, 2 * D), p["mlp_w2"], p["mlp_b2"].reshape(1, D))


def _edge_body(src_hbm, dst3_hbm, typ_hbm, y_hbm, gb_hbm, out_hbm,
               srcblk, typblk, dstblk,
               rows0, rows1, gbv0, gbv1, prod0, prod1, acc,
               semy0, semy1, semg0, semg1, sems0, sems1):
    rows = (rows0, rows1)
    gbv = (gbv0, gbv1)
    prod = (prod0, prod1)
    semy = (semy0, semy1)
    semg = (semg0, semg1)
    sems = (sems0, sems1)

    c = lax.axis_index("c")
    s = lax.axis_index("s")
    wid = c * NS + s
    base = wid * EPW

    def zrow(i, _):
        for j in range(D // 16):
            prod0[i, pl.ds(j * 16, 16)] = jnp.zeros((16,), jnp.float32)
        return 0
    lax.fori_loop(0, K, zrow, 0)
    for k in range(RPS // K):
        pltpu.sync_copy(prod0, acc.at[pl.ds(s * RPS + k * K, K)])
    plsc.subcore_barrier()

    def fire(b, j):
        pltpu.async_copy(y_hbm.at[srcblk.at[pl.ds(j * K, K)]], rows[b],
                         semy[b])
        pltpu.async_copy(gb_hbm.at[typblk.at[pl.ds(j * K, K)]], gbv[b],
                         semg[b])

    def wait_gathers(b):
        pltpu.make_async_copy(y_hbm.at[srcblk.at[pl.ds(0, K)]], rows[b],
                              semy[b]).wait()
        pltpu.make_async_copy(gb_hbm.at[typblk.at[pl.ds(0, K)]], gbv[b],
                              semg[b]).wait()

    def wait_scatter(b):
        pltpu.make_async_copy(prod[b], acc.at[dstblk.at[0]], sems[b]).wait()

    def block(g, _):
        @pl.when(g > 0)
        def _():
            wait_scatter(0)
            wait_scatter(1)
        bb = base + g * EPB
        pltpu.sync_copy(src_hbm.at[pl.ds(bb, EPB)], srcblk)
        pltpu.sync_copy(typ_hbm.at[pl.ds(bb, EPB)], typblk)
        pltpu.sync_copy(dst3_hbm.at[wid * NBLK + g], dstblk)
        fire(0, 0)
        fire(1, 1)

        def pair(pp, _):
            for b in range(2):
                j = 2 * pp + b
                wait_gathers(b)

                @pl.when(pp >= 1)
                def _():
                    wait_scatter(b)

                @plsc.parallel_loop(0, K, unroll=4)
                def erow(e):
                    for q in range(D // 16):
                        sl = pl.ds(q * 16, 16)
                        prod[b][e, sl] = (rows[b][e, sl] * gbv[b][e, sl]
                                          + gbv[b][e, pl.ds(D + q * 16, 16)])
                pltpu.async_copy(prod[b], acc.at[dstblk.at[j]], sems[b],
                                 add=True)

                @pl.when(pp <= CPB // 2 - 2)
                def _():
                    fire(b, j + 2)
            return 0
        lax.fori_loop(0, CPB // 2, pair, 0)
        return 0
    lax.fori_loop(0, NBLK, block, 0)
    wait_scatter(0)
    wait_scatter(1)

    plsc.subcore_barrier()
    for k in range(RPS // K):
        r0 = s * RPS + k * K
        pltpu.sync_copy(acc.at[pl.ds(r0, K)], out_hbm.at[c, pl.ds(r0, K)])


@functools.partial(jax.jit, static_argnums=())
def _edge_pass(src, dst3, typ, y, gb):
    f = pl.kernel(
        _edge_body,
        out_type=jax.ShapeDtypeStruct((NC, NP, D), jnp.float32),
        mesh=plsc.VectorSubcoreMesh(core_axis_name="c", subcore_axis_name="s"),
        scratch_types=[
            pltpu.VMEM((EPB,), jnp.int32),
            pltpu.VMEM((EPB,), jnp.int32),
            pltpu.VMEM((CPB, K), jnp.int32),
            pltpu.VMEM((K, D), jnp.float32),
            pltpu.VMEM((K, D), jnp.float32),
            pltpu.VMEM((K, 2 * D), jnp.float32),
            pltpu.VMEM((K, 2 * D), jnp.float32),
            pltpu.VMEM((K, D), jnp.float32),
            pltpu.VMEM((K, D), jnp.float32),
            pltpu.VMEM_SHARED((NP, D), jnp.float32),
            pltpu.SemaphoreType.DMA,
            pltpu.SemaphoreType.DMA,
            pltpu.SemaphoreType.DMA,
            pltpu.SemaphoreType.DMA,
            pltpu.SemaphoreType.DMA,
            pltpu.SemaphoreType.DMA,
        ],
    )
    return f(src, dst3, typ, y, gb)


def kernel(edge_index, edge_type, embed_w, params):
    src = edge_index[0]
    dst3 = edge_index[1].reshape(NW * NBLK, CPB, K)
    x = embed_w
    for p in params:
        gb = _gb_tables(p)
        y = _linear(x, p["W_w"], p["W_b"])
        part = _edge_pass(src, dst3, edge_type, y, gb)
        x = _update(x, part[0, :N], part[1, :N], p)
    return x


# speedup vs baseline: 4.1320x; 1.1168x over previous
import functools

import jax
import jax.numpy as jnp
from jax import lax
from jax.experimental import pallas as pl
from jax.experimental.pallas import tpu as pltpu
from jax.experimental.pallas import tpu_sc as plsc


N = 10000
E = 320000
D = 128
R = 64

NC = 2
NS = 16
NW = NC * NS
EPW = E // NW
K = 40
NCH = EPW // K
NP = 10240
RPS = NP // NS
NBLK = 25
CPB = 10
EPB = CPB * K

_NBLK = 10
_BN = N // _NBLK


def _gb_kernel(emb_ref, w1_ref, b1_ref, w2_ref, b2_ref, out_ref):
    h = jnp.maximum(
        jnp.dot(emb_ref[...], w1_ref[...], preferred_element_type=jnp.float32)
        + b1_ref[...], 0.0)
    out_ref[...] = (
        jnp.dot(h, w2_ref[...], preferred_element_type=jnp.float32)
        + b2_ref[...])


def _gb_tables(p):
    return pl.pallas_call(
        _gb_kernel,
        out_shape=jax.ShapeDtypeStruct((R, 2 * D), jnp.float32),
    )(p["rel_emb"], p["rel_w1"], p["rel_b1"].reshape(1, 2 * D),
      p["rel_w2"], p["rel_b2"].reshape(1, 2 * D))


def _lin_kernel(x_ref, w_ref, b_ref, out_ref):
    out_ref[...] = (
        jnp.dot(x_ref[...], w_ref[...], preferred_element_type=jnp.float32)
        + b_ref[...])


def _linear(x, w, b):
    return pl.pallas_call(
        _lin_kernel,
        grid=(_NBLK,),
        in_specs=[
            pl.BlockSpec((_BN, D), lambda i: (i, 0)),
            pl.BlockSpec((D, D), lambda i: (0, 0)),
            pl.BlockSpec((1, D), lambda i: (0, 0)),
        ],
        out_specs=pl.BlockSpec((_BN, D), lambda i: (i, 0)),
        out_shape=jax.ShapeDtypeStruct((N, D), jnp.float32),
    )(x, w, b.reshape(1, D))


def _upd_kernel(x_ref, a0_ref, a1_ref, c0_ref, c1_ref, bt_ref, eps_ref,
                w1_ref, b1_ref, w2_ref, b2_ref, out_ref):
    cb = jnp.dot(c0_ref[...] + c1_ref[...], bt_ref[...],
                 preferred_element_type=jnp.float32)
    z = (1.0 + eps_ref[0, 0]) * x_ref[...] + a0_ref[...] + a1_ref[...] + cb
    h = jnp.maximum(
        jnp.dot(z, w1_ref[...], preferred_element_type=jnp.float32)
        + b1_ref[...], 0.0)
    out_ref[...] = (
        jnp.dot(h, w2_ref[...], preferred_element_type=jnp.float32)
        + b2_ref[...])


def _update(x, a0, a1, c0, c1, beta_tbl, p):
    return pl.pallas_call(
        _upd_kernel,
        grid=(_NBLK,),
        in_specs=[
            pl.BlockSpec((_BN, D), lambda i: (i, 0)),
            pl.BlockSpec((_BN, D), lambda i: (i, 0)),
            pl.BlockSpec((_BN, D), lambda i: (i, 0)),
            pl.BlockSpec((_BN, R), lambda i: (i, 0)),
            pl.BlockSpec((_BN, R), lambda i: (i, 0)),
            pl.BlockSpec((R, D), lambda i: (0, 0)),
            pl.BlockSpec((1, 1), lambda i: (0, 0)),
            pl.BlockSpec((D, 2 * D), lambda i: (0, 0)),
            pl.BlockSpec((1, 2 * D), lambda i: (0, 0)),
            pl.BlockSpec((2 * D, D), lambda i: (0, 0)),
            pl.BlockSpec((1, D), lambda i: (0, 0)),
        ],
        out_specs=pl.BlockSpec((_BN, D), lambda i: (i, 0)),
        out_shape=jax.ShapeDtypeStruct((N, D), jnp.float32),
    )(x, a0, a1, c0, c1, beta_tbl, p["eps"].reshape(1, 1), p["mlp_w1"],
      p["mlp_b1"].reshape(1, 2 * D), p["mlp_w2"], p["mlp_b2"].reshape(1, D))


def _edge_body(src_hbm, dst3_hbm, typ_hbm, y_hbm, gb_hbm, out_hbm,
               srcblk, typblk, dstblk,
               rows0, rows1, gbv0, gbv1, prod0, prod1, acc,
               semy0, semy1, semg0, semg1, sems0, sems1):
    rows = (rows0, rows1)
    gbv = (gbv0, gbv1)
    prod = (prod0, prod1)
    semy = (semy0, semy1)
    semg = (semg0, semg1)
    sems = (sems0, sems1)

    c = lax.axis_index("c")
    s = lax.axis_index("s")
    wid = c * NS + s
    base = wid * EPW

    def zrow(i, _):
        for j in range(D // 16):
            prod0[i, pl.ds(j * 16, 16)] = jnp.zeros((16,), jnp.float32)
        return 0
    lax.fori_loop(0, K, zrow, 0)
    for k in range(RPS // K):
        pltpu.sync_copy(prod0, acc.at[pl.ds(s * RPS + k * K, K)])
    plsc.subcore_barrier()

    def fire(b, j):
        pltpu.async_copy(y_hbm.at[srcblk.at[pl.ds(j * K, K)]], rows[b],
                         semy[b])
        pltpu.async_copy(gb_hbm.at[typblk.at[pl.ds(j * K, K)]], gbv[b],
                         semg[b])

    def wait_gathers(b):
        pltpu.make_async_copy(y_hbm.at[srcblk.at[pl.ds(0, K)]], rows[b],
                              semy[b]).wait()
        pltpu.make_async_copy(gb_hbm.at[typblk.at[pl.ds(0, K)]], gbv[b],
                              semg[b]).wait()

    def wait_scatter(b):
        pltpu.make_async_copy(prod[b], acc.at[dstblk.at[0]], sems[b]).wait()

    def block(g, _):
        @pl.when(g > 0)
        def _():
            wait_scatter(0)
            wait_scatter(1)
        bb = base + g * EPB
        pltpu.sync_copy(src_hbm.at[pl.ds(bb, EPB)], srcblk)
        pltpu.sync_copy(typ_hbm.at[pl.ds(bb, EPB)], typblk)
        pltpu.sync_copy(dst3_hbm.at[wid * NBLK + g], dstblk)
        fire(0, 0)
        fire(1, 1)

        def pair(pp, _):
            for b in range(2):
                j = 2 * pp + b
                wait_gathers(b)

                @pl.when(pp >= 1)
                def _():
                    wait_scatter(b)

                @plsc.parallel_loop(0, K, unroll=4)
                def erow(e):
                    for q in range(D // 16):
                        sl = pl.ds(q * 16, 16)
                        prod[b][e, sl] = rows[b][e, sl] * gbv[b][e, sl]
                pltpu.async_copy(prod[b], acc.at[dstblk.at[j]], sems[b],
                                 add=True)

                @pl.when(pp <= CPB // 2 - 2)
                def _():
                    fire(b, j + 2)
            return 0
        lax.fori_loop(0, CPB // 2, pair, 0)
        return 0
    lax.fori_loop(0, NBLK, block, 0)
    wait_scatter(0)
    wait_scatter(1)

    plsc.subcore_barrier()
    for k in range(RPS // K):
        r0 = s * RPS + k * K
        pltpu.sync_copy(acc.at[pl.ds(r0, K)], out_hbm.at[c, pl.ds(r0, K)])


@functools.partial(jax.jit, static_argnums=())
def _edge_pass(src, dst3, typ, y, gb):
    f = pl.kernel(
        _edge_body,
        out_type=jax.ShapeDtypeStruct((NC, NP, D), jnp.float32),
        mesh=plsc.VectorSubcoreMesh(core_axis_name="c", subcore_axis_name="s"),
        scratch_types=[
            pltpu.VMEM((EPB,), jnp.int32),
            pltpu.VMEM((EPB,), jnp.int32),
            pltpu.VMEM((CPB, K), jnp.int32),
            pltpu.VMEM((K, D), jnp.float32),
            pltpu.VMEM((K, D), jnp.float32),
            pltpu.VMEM((K, D), jnp.float32),
            pltpu.VMEM((K, D), jnp.float32),
            pltpu.VMEM((K, D), jnp.float32),
            pltpu.VMEM((K, D), jnp.float32),
            pltpu.VMEM_SHARED((NP, D), jnp.float32),
            pltpu.SemaphoreType.DMA,
            pltpu.SemaphoreType.DMA,
            pltpu.SemaphoreType.DMA,
            pltpu.SemaphoreType.DMA,
            pltpu.SemaphoreType.DMA,
            pltpu.SemaphoreType.DMA,
        ],
    )
    return f(src, dst3, typ, y, gb)


K2 = 80
G2 = 400
NG2 = EPW // G2
CSZ = N * R
CPS = CSZ // NS
ZW = 8000


def _cnt_body(dst_hbm, typ_hbm, out_hbm, dstb, typb, ones, zbuf,
              cx0, cx1, cx2, cx3, cx4, cnt, s0, s1, s2, s3, s4):
    cx = (cx0, cx1, cx2, cx3, cx4)
    sem = (s0, s1, s2, s3, s4)
    c = lax.axis_index("c")
    s = lax.axis_index("s")
    base = (c * NS + s) * EPW

    @plsc.parallel_loop(0, K2 // 16, unroll=2)
    def fill1(q):
        ones[pl.ds(q * 16, 16)] = jnp.full((16,), 1.0, jnp.float32)

    @plsc.parallel_loop(0, ZW // 16, unroll=4)
    def fillz(q):
        zbuf[pl.ds(q * 16, 16)] = jnp.zeros((16,), jnp.float32)
    for k in range(CPS // ZW):
        pltpu.sync_copy(zbuf, cnt.at[pl.ds(s * CPS + k * ZW, ZW)])
    plsc.subcore_barrier()

    def group(gg, _):
        pltpu.sync_copy(dst_hbm.at[pl.ds(base + gg * G2, G2)], dstb)
        pltpu.sync_copy(typ_hbm.at[pl.ds(base + gg * G2, G2)], typb)
        for t in range(5):
            @pl.when(gg >= 1)
            def _():
                pltpu.make_async_copy(ones, cnt.at[cx[t]], sem[t]).wait()

            @plsc.parallel_loop(0, K2 // 16, unroll=2)
            def mkidx(q):
                sl = pl.ds(t * K2 + q * 16, 16)
                cx[t][pl.ds(q * 16, 16)] = (dstb[sl] << 6) + typb[sl]
            pltpu.async_copy(ones, cnt.at[cx[t]], sem[t], add=True)
        return 0
    lax.fori_loop(0, NG2, group, 0)
    for t in range(5):
        pltpu.make_async_copy(ones, cnt.at[cx[t]], sem[t]).wait()

    plsc.subcore_barrier()
    for k in range(CPS // ZW):
        r0 = s * CPS + k * ZW
        pltpu.sync_copy(cnt.at[pl.ds(r0, ZW)], zbuf)
        pltpu.sync_copy(zbuf, out_hbm.at[pl.ds(c * CSZ + r0, ZW)])


def _count_pass(dst, typ):
    f = pl.kernel(
        _cnt_body,
        out_type=jax.ShapeDtypeStruct((NC * CSZ,), jnp.float32),
        mesh=plsc.VectorSubcoreMesh(core_axis_name="c", subcore_axis_name="s"),
        scratch_types=[
            pltpu.VMEM((G2,), jnp.int32),
            pltpu.VMEM((G2,), jnp.int32),
            pltpu.VMEM((K2,), jnp.float32),
            pltpu.VMEM((ZW,), jnp.float32),
            pltpu.VMEM((K2,), jnp.int32),
            pltpu.VMEM((K2,), jnp.int32),
            pltpu.VMEM((K2,), jnp.int32),
            pltpu.VMEM((K2,), jnp.int32),
            pltpu.VMEM((K2,), jnp.int32),
            pltpu.VMEM_SHARED((CSZ,), jnp.float32),
            pltpu.SemaphoreType.DMA,
            pltpu.SemaphoreType.DMA,
            pltpu.SemaphoreType.DMA,
            pltpu.SemaphoreType.DMA,
            pltpu.SemaphoreType.DMA,
        ],
    )
    return f(dst, typ)


def kernel(edge_index, edge_type, embed_w, params):
    src = edge_index[0]
    dst = edge_index[1]
    dst3 = dst.reshape(NW * NBLK, CPB, K)
    cnt = _count_pass(dst, edge_type).reshape(NC, N, R)
    c0 = cnt[0]
    c1 = cnt[1]
    x = embed_w
    for p in params:
        gb = _gb_tables(p)
        gamma_tbl = gb[:, :D]
        beta_tbl = gb[:, D:]
        y = _linear(x, p["W_w"], p["W_b"])
        part = _edge_pass(src, dst3, edge_type, y, gamma_tbl)
        x = _update(x, part[0, :N], part[1, :N], c0, c1, beta_tbl, p)
    return x


# speedup vs baseline: 4.1571x; 1.0061x over previous
import functools

import jax
import jax.numpy as jnp
from jax import lax
from jax.experimental import pallas as pl
from jax.experimental.pallas import tpu as pltpu
from jax.experimental.pallas import tpu_sc as plsc


N = 10000
E = 320000
D = 128
R = 64

NC = 2
NS = 16
NW = NC * NS
EPW = E // NW
K = 40
NCH = EPW // K
NP = 10240
RPS = NP // NS
NBLK = 25
CPB = 10
EPB = CPB * K

_NBLK = 10
_BN = N // _NBLK


def _gb_kernel(emb_ref, w1_ref, b1_ref, w2_ref, b2_ref, out_ref):
    h = jnp.maximum(
        jnp.dot(emb_ref[...], w1_ref[...], preferred_element_type=jnp.float32)
        + b1_ref[...], 0.0)
    out_ref[...] = (
        jnp.dot(h, w2_ref[...], preferred_element_type=jnp.float32)
        + b2_ref[...])


def _gb_tables(p):
    return pl.pallas_call(
        _gb_kernel,
        out_shape=jax.ShapeDtypeStruct((R, 2 * D), jnp.float32),
    )(p["rel_emb"], p["rel_w1"], p["rel_b1"].reshape(1, 2 * D),
      p["rel_w2"], p["rel_b2"].reshape(1, 2 * D))


def _lin_kernel(x_ref, w_ref, b_ref, out_ref):
    out_ref[...] = (
        jnp.dot(x_ref[...], w_ref[...], preferred_element_type=jnp.float32)
        + b_ref[...])


def _linear(x, w, b):
    return pl.pallas_call(
        _lin_kernel,
        grid=(_NBLK,),
        in_specs=[
            pl.BlockSpec((_BN, D), lambda i: (i, 0)),
            pl.BlockSpec((D, D), lambda i: (0, 0)),
            pl.BlockSpec((1, D), lambda i: (0, 0)),
        ],
        out_specs=pl.BlockSpec((_BN, D), lambda i: (i, 0)),
        out_shape=jax.ShapeDtypeStruct((N, D), jnp.float32),
    )(x, w, b.reshape(1, D))


def _upd_kernel(x_ref, a0_ref, a1_ref, c0_ref, c1_ref, bt_ref, eps_ref,
                w1_ref, b1_ref, w2_ref, b2_ref, out_ref):
    cb = jnp.dot(c0_ref[...] + c1_ref[...], bt_ref[...],
                 preferred_element_type=jnp.float32)
    z = (1.0 + eps_ref[0, 0]) * x_ref[...] + a0_ref[...] + a1_ref[...] + cb
    h = jnp.maximum(
        jnp.dot(z, w1_ref[...], preferred_element_type=jnp.float32)
        + b1_ref[...], 0.0)
    out_ref[...] = (
        jnp.dot(h, w2_ref[...], preferred_element_type=jnp.float32)
        + b2_ref[...])


def _update(x, a0, a1, c0, c1, beta_tbl, p):
    return pl.pallas_call(
        _upd_kernel,
        grid=(_NBLK,),
        in_specs=[
            pl.BlockSpec((_BN, D), lambda i: (i, 0)),
            pl.BlockSpec((_BN, D), lambda i: (i, 0)),
            pl.BlockSpec((_BN, D), lambda i: (i, 0)),
            pl.BlockSpec((_BN, R), lambda i: (i, 0)),
            pl.BlockSpec((_BN, R), lambda i: (i, 0)),
            pl.BlockSpec((R, D), lambda i: (0, 0)),
            pl.BlockSpec((1, 1), lambda i: (0, 0)),
            pl.BlockSpec((D, 2 * D), lambda i: (0, 0)),
            pl.BlockSpec((1, 2 * D), lambda i: (0, 0)),
            pl.BlockSpec((2 * D, D), lambda i: (0, 0)),
            pl.BlockSpec((1, D), lambda i: (0, 0)),
        ],
        out_specs=pl.BlockSpec((_BN, D), lambda i: (i, 0)),
        out_shape=jax.ShapeDtypeStruct((N, D), jnp.float32),
    )(x, a0, a1, c0, c1, beta_tbl, p["eps"].reshape(1, 1), p["mlp_w1"],
      p["mlp_b1"].reshape(1, 2 * D), p["mlp_w2"], p["mlp_b2"].reshape(1, D))


def _edge_body(src_hbm, dst3_hbm, typ_hbm, y_hbm, gam_hbm, out_hbm,
               srcblk0, srcblk1, typblk0, typblk1, dstblk0, dstblk1,
               rows0, rows1, rows2, rows3, rows4, gam0, gam1, acc,
               sy0, sy1, sy2, sy3, sy4, sg0, sg1,
               ss0, ss1, ss2, ss3, ss4):
    rows = (rows0, rows1, rows2, rows3, rows4)
    gam = (gam0, gam1)
    sblk = (srcblk0, srcblk1)
    tblk = (typblk0, typblk1)
    dblk = (dstblk0, dstblk1)
    semy = (sy0, sy1, sy2, sy3, sy4)
    semg = (sg0, sg1)
    sems = (ss0, ss1, ss2, ss3, ss4)

    c = lax.axis_index("c")
    s = lax.axis_index("s")
    wid = c * NS + s
    base = wid * EPW

    def zrow(i, _):
        for j in range(D // 16):
            rows0[i, pl.ds(j * 16, 16)] = jnp.zeros((16,), jnp.float32)
        return 0
    lax.fori_loop(0, K, zrow, 0)
    for k in range(RPS // K):
        pltpu.sync_copy(rows0, acc.at[pl.ds(s * RPS + k * K, K)])
    plsc.subcore_barrier()

    def fire_y(b, sb, off):
        pltpu.async_copy(y_hbm.at[sb.at[pl.ds(off, K)]], rows[b], semy[b])

    def fire_g(b, tb, off):
        pltpu.async_copy(gam_hbm.at[tb.at[pl.ds(off, K)]], gam[b], semg[b])

    def wait_y(b):
        pltpu.make_async_copy(y_hbm.at[sblk[0].at[pl.ds(0, K)]], rows[b],
                              semy[b]).wait()

    def wait_g(b):
        pltpu.make_async_copy(gam_hbm.at[tblk[0].at[pl.ds(0, K)]], gam[b],
                              semg[b]).wait()

    def wait_s(b):
        pltpu.make_async_copy(rows[b], acc.at[dstblk0.at[0]], sems[b]).wait()

    def slot(g, t, pg, tail):
        br = t % 5
        bg = t % 2
        wb = (t - 2) % 5
        if t >= 2:
            wait_s(wb)
        else:
            @pl.when(g > 0)
            def _():
                wait_s(wb)
        if not (tail and t >= 7):
            yt = (t + 3) % 5
            fire_y(yt, sblk[pg] if t <= 6 else sblk[1 - pg],
                   ((t + 3) % 10) * K)
        wait_y(br)
        wait_g(bg)

        @plsc.parallel_loop(0, K, unroll=2)
        def erow(e):
            for q in range(D // 16):
                sl = pl.ds(q * 16, 16)
                rows[br][e, sl] = rows[br][e, sl] * gam[bg][e, sl]
        pltpu.async_copy(rows[br], acc.at[dblk[pg].at[t]], sems[br],
                         add=True)
        if not (tail and t >= 8):
            fire_g(bg, tblk[pg] if t <= 7 else tblk[1 - pg],
                   ((t + 2) % 10) * K)
        if (not tail) and t == 2:
            nb = base + (g + 1) * EPB
            pltpu.sync_copy(src_hbm.at[pl.ds(nb, EPB)], sblk[1 - pg])
            pltpu.sync_copy(typ_hbm.at[pl.ds(nb, EPB)], tblk[1 - pg])
            pltpu.sync_copy(dst3_hbm.at[wid * NBLK + g + 1], dblk[1 - pg])

    pltpu.sync_copy(src_hbm.at[pl.ds(base, EPB)], srcblk0)
    pltpu.sync_copy(typ_hbm.at[pl.ds(base, EPB)], typblk0)
    pltpu.sync_copy(dst3_hbm.at[wid * NBLK], dstblk0)
    for j in range(3):
        fire_y(j, srcblk0, j * K)
    for j in range(2):
        fire_g(j, typblk0, j * K)

    def pairblk(gg, _):
        for gp in range(2):
            g = 2 * gg + gp
            for t in range(CPB):
                slot(g, t, gp, False)
        return 0
    lax.fori_loop(0, (NBLK - 1) // 2, pairblk, 0)
    for t in range(CPB):
        slot(NBLK - 1, t, 0, True)
    wait_s(3)
    wait_s(4)

    plsc.subcore_barrier()
    for k in range(RPS // K):
        r0 = s * RPS + k * K
        pltpu.sync_copy(acc.at[pl.ds(r0, K)], out_hbm.at[c, pl.ds(r0, K)])


@functools.partial(jax.jit, static_argnums=())
def _edge_pass(src, dst3, typ, y, gam_tbl):
    f = pl.kernel(
        _edge_body,
        out_type=jax.ShapeDtypeStruct((NC, NP, D), jnp.float32),
        mesh=plsc.VectorSubcoreMesh(core_axis_name="c", subcore_axis_name="s"),
        scratch_types=[
            pltpu.VMEM((EPB,), jnp.int32),
            pltpu.VMEM((EPB,), jnp.int32),
            pltpu.VMEM((EPB,), jnp.int32),
            pltpu.VMEM((EPB,), jnp.int32),
            pltpu.VMEM((CPB, K), jnp.int32),
            pltpu.VMEM((CPB, K), jnp.int32),
            pltpu.VMEM((K, D), jnp.float32),
            pltpu.VMEM((K, D), jnp.float32),
            pltpu.VMEM((K, D), jnp.float32),
            pltpu.VMEM((K, D), jnp.float32),
            pltpu.VMEM((K, D), jnp.float32),
            pltpu.VMEM((K, D), jnp.float32),
            pltpu.VMEM((K, D), jnp.float32),
            pltpu.VMEM_SHARED((NP, D), jnp.float32),
            pltpu.SemaphoreType.DMA,
            pltpu.SemaphoreType.DMA,
            pltpu.SemaphoreType.DMA,
            pltpu.SemaphoreType.DMA,
            pltpu.SemaphoreType.DMA,
            pltpu.SemaphoreType.DMA,
            pltpu.SemaphoreType.DMA,
            pltpu.SemaphoreType.DMA,
            pltpu.SemaphoreType.DMA,
            pltpu.SemaphoreType.DMA,
            pltpu.SemaphoreType.DMA,
            pltpu.SemaphoreType.DMA,
        ],
    )
    return f(src, dst3, typ, y, gam_tbl)


K2 = 80
G2 = 400
NG2 = EPW // G2
CSZ = N * R
CPS = CSZ // NS
ZW = 8000


def _cnt_body(dst_hbm, typ_hbm, out_hbm, dstb, typb, ones, zbuf,
              cx0, cx1, cx2, cx3, cx4, cnt, s0, s1, s2, s3, s4):
    cx = (cx0, cx1, cx2, cx3, cx4)
    sem = (s0, s1, s2, s3, s4)
    c = lax.axis_index("c")
    s = lax.axis_index("s")
    base = (c * NS + s) * EPW

    @plsc.parallel_loop(0, K2 // 16, unroll=2)
    def fill1(q):
        ones[pl.ds(q * 16, 16)] = jnp.full((16,), 1.0, jnp.float32)

    @plsc.parallel_loop(0, ZW // 16, unroll=4)
    def fillz(q):
        zbuf[pl.ds(q * 16, 16)] = jnp.zeros((16,), jnp.float32)
    for k in range(CPS // ZW):
        pltpu.sync_copy(zbuf, cnt.at[pl.ds(s * CPS + k * ZW, ZW)])
    plsc.subcore_barrier()

    def group(gg, _):
        pltpu.sync_copy(dst_hbm.at[pl.ds(base + gg * G2, G2)], dstb)
        pltpu.sync_copy(typ_hbm.at[pl.ds(base + gg * G2, G2)], typb)
        for t in range(5):
            @pl.when(gg >= 1)
            def _():
                pltpu.make_async_copy(ones, cnt.at[cx[t]], sem[t]).wait()

            @plsc.parallel_loop(0, K2 // 16, unroll=2)
            def mkidx(q):
                sl = pl.ds(t * K2 + q * 16, 16)
                cx[t][pl.ds(q * 16, 16)] = (dstb[sl] << 6) + typb[sl]
            pltpu.async_copy(ones, cnt.at[cx[t]], sem[t], add=True)
        return 0
    lax.fori_loop(0, NG2, group, 0)
    for t in range(5):
        pltpu.make_async_copy(ones, cnt.at[cx[t]], sem[t]).wait()

    plsc.subcore_barrier()
    for k in range(CPS // ZW):
        r0 = s * CPS + k * ZW
        pltpu.sync_copy(cnt.at[pl.ds(r0, ZW)], zbuf)
        pltpu.sync_copy(zbuf, out_hbm.at[pl.ds(c * CSZ + r0, ZW)])


def _count_pass(dst, typ):
    f = pl.kernel(
        _cnt_body,
        out_type=jax.ShapeDtypeStruct((NC * CSZ,), jnp.float32),
        mesh=plsc.VectorSubcoreMesh(core_axis_name="c", subcore_axis_name="s"),
        scratch_types=[
            pltpu.VMEM((G2,), jnp.int32),
            pltpu.VMEM((G2,), jnp.int32),
            pltpu.VMEM((K2,), jnp.float32),
            pltpu.VMEM((ZW,), jnp.float32),
            pltpu.VMEM((K2,), jnp.int32),
            pltpu.VMEM((K2,), jnp.int32),
            pltpu.VMEM((K2,), jnp.int32),
            pltpu.VMEM((K2,), jnp.int32),
            pltpu.VMEM((K2,), jnp.int32),
            pltpu.VMEM_SHARED((CSZ,), jnp.float32),
            pltpu.SemaphoreType.DMA,
            pltpu.SemaphoreType.DMA,
            pltpu.SemaphoreType.DMA,
            pltpu.SemaphoreType.DMA,
            pltpu.SemaphoreType.DMA,
        ],
    )
    return f(dst, typ)


def kernel(edge_index, edge_type, embed_w, params):
    src = edge_index[0]
    dst = edge_index[1]
    dst3 = dst.reshape(NW * NBLK, CPB, K)
    cnt = _count_pass(dst, edge_type).reshape(NC, N, R)
    c0 = cnt[0]
    c1 = cnt[1]
    x = embed_w
    for p in params:
        gb = _gb_tables(p)
        gamma_tbl = gb[:, :D]
        beta_tbl = gb[:, D:]
        y = _linear(x, p["W_w"], p["W_b"])
        part = _edge_pass(src, dst3, edge_type, y, gamma_tbl)
        x = _update(x, part[0, :N], part[1, :N], c0, c1, beta_tbl, p)
    return x
